# baseline (speedup 1.0000x reference)
import functools

import jax
import jax.numpy as jnp
from jax import lax
from jax.experimental import pallas as pl
from jax.experimental.pallas import tpu as pltpu
from jax.experimental.pallas import tpu_sc as plsc

_N = 320000
_F = 128
_H = 128
_C = 64
_B = 10000

_R = 1280
_NTILES = 32
_CHUNK = 1024
_NPAD = 327680
_ROWS_PER_TILE = _NPAD // _NTILES
_NCHUNKS = _ROWS_PER_TILE // _CHUNK
_NS = 16
_BP = 10240
_SEG_PER_TILE = _BP // _NS
_CP = 128


def _producer_body(x_ref, s_ref, w1_ref, w2_ref, out_ref, *, n_real_blocks):
    i = pl.program_id(0)

    @pl.when(i < n_real_blocks)
    def _compute():
        h = jnp.maximum(
            jnp.dot(x_ref[...].astype(jnp.bfloat16), w1_ref[...],
                    preferred_element_type=jnp.float32),
            0.0)
        logits = jnp.dot(h.astype(jnp.bfloat16), w2_ref[...],
                         preferred_element_type=jnp.float32)
        out_ref[...] = logits * s_ref[...]

    @pl.when(i >= n_real_blocks)
    def _pad():
        out_ref[...] = jnp.zeros_like(out_ref)


def _producer(X, scores2d, W1, W2):
    n_real_blocks = _N // _R
    grid = _NPAD // _R
    clamp = n_real_blocks - 1
    body = functools.partial(_producer_body, n_real_blocks=n_real_blocks)
    return pl.pallas_call(
        body,
        grid=(grid,),
        in_specs=[
            pl.BlockSpec((_R, _F), lambda i: (jnp.minimum(i, clamp), 0)),
            pl.BlockSpec((_R, 1), lambda i: (jnp.minimum(i, clamp), 0)),
            pl.BlockSpec((_F, _H), lambda i: (0, 0)),
            pl.BlockSpec((_H, _CP), lambda i: (0, 0)),
        ],
        out_specs=pl.BlockSpec((_R, _CP), lambda i: (i, 0)),
        out_shape=jax.ShapeDtypeStruct((_NPAD, _CP), jnp.float32),
    )(X, scores2d, W1.astype(jnp.bfloat16),
      jnp.pad(W2, ((0, 0), (0, _CP - _C))).astype(jnp.bfloat16))


def _sc_scatter_body(wtd_hbm, idx_hbm, iota_hbm, zero_hbm, out_hbm,
                     rows_v, idx_v, iota_v, acc_sh):
    cid = lax.axis_index("c")
    sid = lax.axis_index("s")
    wid = cid * _NS + sid

    pltpu.sync_copy(iota_hbm.at[pl.ds(sid * 8, 8)], iota_v)
    pltpu.sync_copy(zero_hbm, rows_v)

    for r in range(5):
        pltpu.sync_copy(rows_v, acc_sh.at[iota_v.at[r]])
    plsc.subcore_barrier()

    base = wid * _ROWS_PER_TILE

    def group_body(k, _):
        off = base + k * 1024
        idx_off = pl.multiple_of(off // 128, 8)
        pltpu.sync_copy(idx_hbm.at[pl.ds(idx_off, 8)], idx_v)
        for j in range(8):
            pltpu.sync_copy(wtd_hbm.at[pl.ds(off + j * 128, 128)], rows_v)
            pltpu.sync_copy(rows_v, acc_sh.at[idx_v.at[j]], add=True)
        return 0

    lax.fori_loop(0, _ROWS_PER_TILE // 1024, group_body, 0)
    plsc.subcore_barrier()

    for r in range(5):
        pltpu.sync_copy(acc_sh.at[iota_v.at[r]], rows_v)
        pltpu.sync_copy(
            rows_v,
            out_hbm.at[cid, pl.ds(sid * _SEG_PER_TILE + r * 128, 128)])


_sc_scatter = pl.kernel(
    _sc_scatter_body,
    mesh=plsc.VectorSubcoreMesh(core_axis_name="c", subcore_axis_name="s"),
    out_type=jax.ShapeDtypeStruct((2, _BP, _CP), jnp.float32),
    scratch_types=[
        pltpu.VMEM((128, _CP), jnp.float32),
        pltpu.VMEM((8, 128), jnp.int32),
        pltpu.VMEM((8, 128), jnp.int32),
        pltpu.VMEM_SHARED((_BP, _CP), jnp.float32),
    ],
)


def _combine_body(p_ref, o_ref):
    o_ref[...] = p_ref[0, 0:_B, 0:_C] + p_ref[1, 0:_B, 0:_C]


def _combine(partials):
    return pl.pallas_call(
        _combine_body,
        grid=(1,),
        in_specs=[pl.BlockSpec((2, _BP, _CP), lambda i: (0, 0, 0))],
        out_specs=pl.BlockSpec((_B, _C), lambda i: (0, 0)),
        out_shape=jax.ShapeDtypeStruct((_B, _C), jnp.float32),
    )(partials)


def _flush_iota():
    g = jnp.arange(128, dtype=jnp.int32)[:, None]
    lane = jnp.arange(128, dtype=jnp.int32)[None, :]
    sid, r = g // 8, g % 8
    return jnp.where(r < 5, sid * _SEG_PER_TILE + r * 128 + lane, 0)


def kernel(X, ppr_scores, ppr_idx, W1, W2):
    scores2d = ppr_scores.reshape(_N, 1)
    wtd = _producer(X, scores2d, W1, W2)
    idx_pad = jnp.pad(ppr_idx, (0, _NPAD - _N)).reshape(_NPAD // 128, 128)
    iota = _flush_iota()
    zeros = jnp.zeros((128, _CP), jnp.float32)
    partials = _sc_scatter(wtd, idx_pad, iota, zeros)
    return _combine(partials)


# speedup vs baseline: 1.0678x; 1.0678x over previous
import functools

import jax
import jax.numpy as jnp
from jax import lax
from jax.experimental import pallas as pl
from jax.experimental.pallas import tpu as pltpu
from jax.experimental.pallas import tpu_sc as plsc

_N = 320000
_F = 128
_H = 128
_C = 64
_B = 10000

_R = 1280
_NTILES = 32
_CHUNK = 1024
_NPAD = 327680
_ROWS_PER_TILE = _NPAD // _NTILES
_NCHUNKS = _ROWS_PER_TILE // _CHUNK
_NS = 16
_BP = 10240
_SEG_PER_TILE = _BP // _NS
_CP = 128


def _producer_body(x_ref, s_ref, w1_ref, w2_ref, out_ref, *, n_real_blocks):
    i = pl.program_id(0)

    @pl.when(i < n_real_blocks)
    def _compute():
        h = jnp.maximum(
            jnp.dot(x_ref[...].astype(jnp.bfloat16), w1_ref[...],
                    preferred_element_type=jnp.float32),
            0.0)
        logits = jnp.dot(h.astype(jnp.bfloat16), w2_ref[...],
                         preferred_element_type=jnp.float32)
        out_ref[...] = logits * s_ref[...]

    @pl.when(i >= n_real_blocks)
    def _pad():
        out_ref[...] = jnp.zeros_like(out_ref)


def _producer(X, scores2d, W1, W2):
    n_real_blocks = _N // _R
    grid = _NPAD // _R
    clamp = n_real_blocks - 1
    body = functools.partial(_producer_body, n_real_blocks=n_real_blocks)
    return pl.pallas_call(
        body,
        grid=(grid,),
        in_specs=[
            pl.BlockSpec((_R, _F), lambda i: (jnp.minimum(i, clamp), 0)),
            pl.BlockSpec((_R, 1), lambda i: (jnp.minimum(i, clamp), 0)),
            pl.BlockSpec((_F, _H), lambda i: (0, 0)),
            pl.BlockSpec((_H, _CP), lambda i: (0, 0)),
        ],
        out_specs=pl.BlockSpec((_R, _CP), lambda i: (i, 0)),
        out_shape=jax.ShapeDtypeStruct((_NPAD, _CP), jnp.float32),
    )(X, scores2d, W1.astype(jnp.bfloat16),
      jnp.pad(W2, ((0, 0), (0, _CP - _C))).astype(jnp.bfloat16))


def _sc_scatter_body(wtd_hbm, idx_hbm, iota_hbm, zero_hbm, out_hbm,
                     rows_v, idx_v, iota_v, acc_sh, ld_sem_a, ld_sem_b):
    cid = lax.axis_index("c")
    sid = lax.axis_index("s")
    wid = cid * _NS + sid

    pltpu.sync_copy(iota_hbm.at[pl.ds(sid * 8, 8)], iota_v)
    pltpu.sync_copy(zero_hbm, rows_v)

    for r in range(5):
        pltpu.sync_copy(rows_v, acc_sh.at[iota_v.at[r]])
    plsc.subcore_barrier()

    base = wid * _ROWS_PER_TILE

    def group_body(k, _):
        off = base + k * 1024
        idx_off = pl.multiple_of(off // 128, 8)
        pltpu.sync_copy(idx_hbm.at[pl.ds(idx_off, 8)], idx_v)
        sems = (ld_sem_a, ld_sem_b)
        pending = pltpu.async_copy(wtd_hbm.at[pl.ds(off, 64)],
                                   rows_v.at[pl.ds(0, 64)], sems[0])
        for j in range(16):
            cur = (j % 2) * 64
            this_copy = pending
            if j < 15:
                nxt = ((j + 1) % 2) * 64
                pending = pltpu.async_copy(
                    wtd_hbm.at[pl.ds(off + (j + 1) * 64, 64)],
                    rows_v.at[pl.ds(nxt, 64)], sems[(j + 1) % 2])
            this_copy.wait()
            pltpu.sync_copy(rows_v.at[pl.ds(cur, 64)],
                            acc_sh.at[idx_v.at[j // 2, pl.ds(cur, 64)]],
                            add=True)
        return 0

    lax.fori_loop(0, _ROWS_PER_TILE // 1024, group_body, 0)
    plsc.subcore_barrier()

    for r in range(5):
        pltpu.sync_copy(acc_sh.at[iota_v.at[r]], rows_v)
        pltpu.sync_copy(
            rows_v,
            out_hbm.at[cid, pl.ds(sid * _SEG_PER_TILE + r * 128, 128)])


_sc_scatter = pl.kernel(
    _sc_scatter_body,
    mesh=plsc.VectorSubcoreMesh(core_axis_name="c", subcore_axis_name="s"),
    out_type=jax.ShapeDtypeStruct((2, _BP, _CP), jnp.float32),
    scratch_types=[
        pltpu.VMEM((128, _CP), jnp.float32),
        pltpu.VMEM((8, 128), jnp.int32),
        pltpu.VMEM((8, 128), jnp.int32),
        pltpu.VMEM_SHARED((_BP, _CP), jnp.float32),
        pltpu.SemaphoreType.DMA,
        pltpu.SemaphoreType.DMA,
    ],
)


def _combine_body(p_ref, o_ref):
    o_ref[...] = p_ref[0, 0:_B, 0:_C] + p_ref[1, 0:_B, 0:_C]


def _combine(partials):
    return pl.pallas_call(
        _combine_body,
        grid=(1,),
        in_specs=[pl.BlockSpec((2, _BP, _CP), lambda i: (0, 0, 0))],
        out_specs=pl.BlockSpec((_B, _C), lambda i: (0, 0)),
        out_shape=jax.ShapeDtypeStruct((_B, _C), jnp.float32),
    )(partials)


def _flush_iota():
    g = jnp.arange(128, dtype=jnp.int32)[:, None]
    lane = jnp.arange(128, dtype=jnp.int32)[None, :]
    sid, r = g // 8, g % 8
    return jnp.where(r < 5, sid * _SEG_PER_TILE + r * 128 + lane, 0)


def kernel(X, ppr_scores, ppr_idx, W1, W2):
    scores2d = ppr_scores.reshape(_N, 1)
    wtd = _producer(X, scores2d, W1, W2)
    idx_pad = jnp.pad(ppr_idx, (0, _NPAD - _N)).reshape(_NPAD // 128, 128)
    iota = _flush_iota()
    zeros = jnp.zeros((128, _CP), jnp.float32)
    partials = _sc_scatter(wtd, idx_pad, iota, zeros)
    return _combine(partials)


# speedup vs baseline: 1.2573x; 1.1775x over previous
import functools

import jax
import jax.numpy as jnp
from jax import lax
from jax.experimental import pallas as pl
from jax.experimental.pallas import tpu as pltpu
from jax.experimental.pallas import tpu_sc as plsc

_N = 320000
_F = 128
_H = 128
_C = 64
_B = 10000

_R = 2560
_NTILES = 32
_CHUNK = 1024
_NPAD = 327680
_ROWS_PER_TILE = _NPAD // _NTILES
_NCHUNKS = _ROWS_PER_TILE // _CHUNK
_NS = 16
_BP = 10240
_SEG_PER_TILE = _BP // _NS
_CP = 128


def _producer_body(x_ref, s_ref, w1_ref, w2_ref, out_ref, *, n_real_blocks):
    i = pl.program_id(0)

    @pl.when(i < n_real_blocks)
    def _compute():
        h = jnp.maximum(
            jnp.dot(x_ref[...].astype(jnp.bfloat16), w1_ref[...],
                    preferred_element_type=jnp.float32),
            0.0)
        logits = jnp.dot(h.astype(jnp.bfloat16), w2_ref[...],
                         preferred_element_type=jnp.float32)
        out_ref[...] = logits * s_ref[...]

    @pl.when(i >= n_real_blocks)
    def _pad():
        out_ref[...] = jnp.zeros_like(out_ref)


def _producer(X, scores2d, W1, W2):
    n_real_blocks = _N // _R
    grid = _NPAD // _R
    clamp = n_real_blocks - 1
    body = functools.partial(_producer_body, n_real_blocks=n_real_blocks)
    return pl.pallas_call(
        body,
        grid=(grid,),
        in_specs=[
            pl.BlockSpec((_R, _F), lambda i: (jnp.minimum(i, clamp), 0)),
            pl.BlockSpec((_R, 1), lambda i: (jnp.minimum(i, clamp), 0)),
            pl.BlockSpec((_F, _H), lambda i: (0, 0)),
            pl.BlockSpec((_H, _CP), lambda i: (0, 0)),
        ],
        out_specs=pl.BlockSpec((_R, _CP), lambda i: (i, 0)),
        out_shape=jax.ShapeDtypeStruct((_NPAD, _CP), jnp.float32),
    )(X, scores2d, W1.astype(jnp.bfloat16),
      jnp.pad(W2, ((0, 0), (0, _CP - _C))).astype(jnp.bfloat16))


def _sc_scatter_body(wtd_hbm, idx_hbm, iota_hbm, zero_hbm, out_hbm,
                     rows_v, idx_v, iota_v, acc_sh, ld_sem_a, ld_sem_b):
    cid = lax.axis_index("c")
    sid = lax.axis_index("s")
    wid = cid * _NS + sid

    pltpu.sync_copy(iota_hbm.at[pl.ds(sid * 8, 8)], iota_v)
    pltpu.sync_copy(zero_hbm, rows_v)

    for r in range(5):
        pltpu.sync_copy(rows_v, acc_sh.at[iota_v.at[r]])
    plsc.subcore_barrier()

    base = wid * _ROWS_PER_TILE

    def group_body(k, _):
        off = base + k * 1024
        idx_off = pl.multiple_of(off // 128, 8)
        pltpu.sync_copy(idx_hbm.at[pl.ds(idx_off, 8)], idx_v)
        sems = (ld_sem_a, ld_sem_b)
        pending = pltpu.async_copy(wtd_hbm.at[pl.ds(off, 64)],
                                   rows_v.at[pl.ds(0, 64)], sems[0])
        for j in range(16):
            cur = (j % 2) * 64
            this_copy = pending
            if j < 15:
                nxt = ((j + 1) % 2) * 64
                pending = pltpu.async_copy(
                    wtd_hbm.at[pl.ds(off + (j + 1) * 64, 64)],
                    rows_v.at[pl.ds(nxt, 64)], sems[(j + 1) % 2])
            this_copy.wait()
            pltpu.sync_copy(rows_v.at[pl.ds(cur, 64)],
                            acc_sh.at[idx_v.at[j // 2, pl.ds(cur, 64)]],
                            add=True)
        return 0

    lax.fori_loop(0, _ROWS_PER_TILE // 1024, group_body, 0)
    plsc.subcore_barrier()

    for r in range(5):
        pltpu.sync_copy(acc_sh.at[iota_v.at[r]], rows_v)
        pltpu.sync_copy(
            rows_v,
            out_hbm.at[cid, pl.ds(sid * _SEG_PER_TILE + r * 128, 128)])


_sc_scatter = pl.kernel(
    _sc_scatter_body,
    mesh=plsc.VectorSubcoreMesh(core_axis_name="c", subcore_axis_name="s"),
    out_type=jax.ShapeDtypeStruct((2, _BP, _CP), jnp.float32),
    scratch_types=[
        pltpu.VMEM((128, _CP), jnp.float32),
        pltpu.VMEM((8, 128), jnp.int32),
        pltpu.VMEM((8, 128), jnp.int32),
        pltpu.VMEM_SHARED((_BP, _CP), jnp.float32),
        pltpu.SemaphoreType.DMA,
        pltpu.SemaphoreType.DMA,
    ],
)


def _combine_body(p_ref, o_ref):
    o_ref[...] = p_ref[0, 0:_B, 0:_C] + p_ref[1, 0:_B, 0:_C]


def _combine(partials):
    return pl.pallas_call(
        _combine_body,
        grid=(1,),
        in_specs=[pl.BlockSpec((2, _BP, _CP), lambda i: (0, 0, 0))],
        out_specs=pl.BlockSpec((_B, _C), lambda i: (0, 0)),
        out_shape=jax.ShapeDtypeStruct((_B, _C), jnp.float32),
    )(partials)


def _flush_iota():
    g = jnp.arange(128, dtype=jnp.int32)[:, None]
    lane = jnp.arange(128, dtype=jnp.int32)[None, :]
    sid, r = g // 8, g % 8
    return jnp.where(r < 5, sid * _SEG_PER_TILE + r * 128 + lane, 0)


def kernel(X, ppr_scores, ppr_idx, W1, W2):
    scores2d = ppr_scores.reshape(_N, 1)
    wtd = _producer(X, scores2d, W1, W2)
    idx_pad = jnp.pad(ppr_idx, (0, _NPAD - _N)).reshape(_NPAD // 128, 128)
    iota = _flush_iota()
    zeros = jnp.zeros((128, _CP), jnp.float32)
    partials = _sc_scatter(wtd, idx_pad, iota, zeros)
    return _combine(partials)


# speedup vs baseline: 1.2595x; 1.0018x over previous
import functools

import jax
import jax.numpy as jnp
from jax import lax
from jax.experimental import pallas as pl
from jax.experimental.pallas import tpu as pltpu
from jax.experimental.pallas import tpu_sc as plsc

_N = 320000
_F = 128
_H = 128
_C = 64
_B = 10000

_R = 2560
_NTILES = 32
_NPAD = 327680
_ROWS_PER_TILE = _NPAD // _NTILES
_NS = 16
_BP = 10240
_SEG_PER_TILE = _BP // _NS
_CP = 128


def _producer_body(x_ref, s_ref, w1_ref, w2_ref, out_ref, *, n_real_blocks):
    i = pl.program_id(0)

    @pl.when(i < n_real_blocks)
    def _compute():
        h = jnp.maximum(
            jnp.dot(x_ref[...].astype(jnp.bfloat16), w1_ref[...],
                    preferred_element_type=jnp.float32),
            0.0)
        logits = jnp.dot(h.astype(jnp.bfloat16), w2_ref[...],
                         preferred_element_type=jnp.float32)
        out_ref[...] = logits * s_ref[...]

    @pl.when(i >= n_real_blocks)
    def _pad():
        out_ref[...] = jnp.zeros_like(out_ref)


def _producer(X, scores2d, W1, W2):
    n_real_blocks = _N // _R
    grid = _NPAD // _R
    clamp = n_real_blocks - 1
    body = functools.partial(_producer_body, n_real_blocks=n_real_blocks)
    return pl.pallas_call(
        body,
        grid=(grid,),
        in_specs=[
            pl.BlockSpec((_R, _F), lambda i: (jnp.minimum(i, clamp), 0)),
            pl.BlockSpec((_R, 1), lambda i: (jnp.minimum(i, clamp), 0)),
            pl.BlockSpec((_F, _H), lambda i: (0, 0)),
            pl.BlockSpec((_H, _CP), lambda i: (0, 0)),
        ],
        out_specs=pl.BlockSpec((_R, _CP), lambda i: (i, 0)),
        out_shape=jax.ShapeDtypeStruct((_NPAD, _CP), jnp.float32),
    )(X, scores2d, W1.astype(jnp.bfloat16),
      jnp.pad(W2, ((0, 0), (0, _CP - _C))).astype(jnp.bfloat16))


def _sc_scatter_body(wtd_hbm, idx_hbm, iota_hbm, zero_hbm, out_hbm,
                     rows_v, idx_v, iota_v, acc_sh, ld_sem_a, ld_sem_b):
    cid = lax.axis_index("c")
    sid = lax.axis_index("s")
    wid = cid * _NS + sid

    pltpu.sync_copy(iota_hbm.at[pl.ds(sid * 8, 8)], iota_v)
    pltpu.sync_copy(zero_hbm, rows_v)

    for r in range(5):
        pltpu.sync_copy(rows_v, acc_sh.at[iota_v.at[r]])
    plsc.subcore_barrier()

    base = wid * _ROWS_PER_TILE

    def group_body(k, _):
        off = base + k * 1024
        idx_off = pl.multiple_of(off // 128, 8)
        pltpu.sync_copy(idx_hbm.at[pl.ds(idx_off, 8)], idx_v)
        sems = (ld_sem_a, ld_sem_b)
        pending = pltpu.async_copy(wtd_hbm.at[pl.ds(off, 64)],
                                   rows_v.at[pl.ds(0, 64)], sems[0])
        for j in range(16):
            cur = (j % 2) * 64
            this_copy = pending
            if j < 15:
                nxt = ((j + 1) % 2) * 64
                pending = pltpu.async_copy(
                    wtd_hbm.at[pl.ds(off + (j + 1) * 64, 64)],
                    rows_v.at[pl.ds(nxt, 64)], sems[(j + 1) % 2])
            this_copy.wait()
            pltpu.sync_copy(rows_v.at[pl.ds(cur, 64)],
                            acc_sh.at[idx_v.at[j // 2, pl.ds(cur, 64)]],
                            add=True)
        return 0

    lax.fori_loop(0, _ROWS_PER_TILE // 1024, group_body, 0)
    plsc.subcore_barrier()

    for r in range(5):
        pltpu.sync_copy(acc_sh.at[iota_v.at[r]], rows_v)
        pltpu.sync_copy(
            rows_v,
            out_hbm.at[cid, pl.ds(sid * _SEG_PER_TILE + r * 128, 128)])


_sc_scatter = pl.kernel(
    _sc_scatter_body,
    mesh=plsc.VectorSubcoreMesh(core_axis_name="c", subcore_axis_name="s"),
    out_type=jax.ShapeDtypeStruct((2, _BP, _CP), jnp.float32),
    scratch_types=[
        pltpu.VMEM((128, _CP), jnp.float32),
        pltpu.VMEM((8, 128), jnp.int32),
        pltpu.VMEM((8, 128), jnp.int32),
        pltpu.VMEM_SHARED((_BP, _CP), jnp.float32),
        pltpu.SemaphoreType.DMA,
        pltpu.SemaphoreType.DMA,
    ],
)


def _combine_body(p_ref, o_ref):
    o_ref[...] = p_ref[0, 0:_B, 0:_C] + p_ref[1, 0:_B, 0:_C]


def _combine(partials):
    return pl.pallas_call(
        _combine_body,
        grid=(1,),
        in_specs=[pl.BlockSpec((2, _BP, _CP), lambda i: (0, 0, 0))],
        out_specs=pl.BlockSpec((_B, _C), lambda i: (0, 0)),
        out_shape=jax.ShapeDtypeStruct((_B, _C), jnp.float32),
    )(partials)


def _flush_iota():
    g = jnp.arange(128, dtype=jnp.int32)[:, None]
    lane = jnp.arange(128, dtype=jnp.int32)[None, :]
    sid, r = g // 8, g % 8
    return jnp.where(r < 5, sid * _SEG_PER_TILE + r * 128 + lane, 0)


def kernel(X, ppr_scores, ppr_idx, W1, W2):
    scores2d = ppr_scores.reshape(_N, 1)
    wtd = _producer(X, scores2d, W1, W2)
    idx_pad = jnp.pad(ppr_idx, (0, _NPAD - _N)).reshape(_NPAD // 128, 128)
    iota = _flush_iota()
    zeros = jnp.zeros((128, _CP), jnp.float32)
    partials = _sc_scatter(wtd, idx_pad, iota, zeros)
    return _combine(partials)
